# Initial kernel scaffold; baseline (speedup 1.0000x reference)
import jax
import jax.numpy as jnp
from jax.experimental import pallas as pl


def kernel(x, edge_index, batch, conv1_w1, conv1_b1, conv1_w2, conv1_b2, convs_w1, convs_b1, convs_w2, convs_b2, bn_gamma, bn_beta, fc1_w, fc1_b, fc2_w, fc2_b):
    raise NotImplementedError("write your pallas kernel here")


# speedup vs baseline: 2.9612x; 2.9612x over previous
import functools

import jax
import jax.numpy as jnp
from jax import lax
from jax.experimental import pallas as pl
from jax.experimental.pallas import tpu as pltpu
from jax.experimental.pallas import tpu_sc as plsc

N = 10000
E = 320000
NG = 64
NCLS = 10

NSUB = 16
EPT = E // NSUB
W = 128
CW = 16
NCHUNK = -(-EPT // (W * CW))
NWIN = NCHUNK * CW
EPT_PAD = NWIN * W
NPAD = EPT_PAD - EPT
NTRASH = 512
NROWS = N + NTRASH


EPT1 = E // (2 * NSUB)
NCHUNK1 = -(-EPT1 // (W * CW))
NWIN1 = NCHUNK1 * CW
NPAD1 = NWIN1 * W - EPT1


@functools.cache
def _make_agg1():
    mesh = plsc.VectorSubcoreMesh(core_axis_name="c", subcore_axis_name="s",
                                  num_cores=2, num_subcores=NSUB)
    rpt = N // NSUB

    @functools.partial(
        pl.kernel,
        mesh=mesh,
        out_type=jax.ShapeDtypeStruct((2 * NSUB, rpt, 128), jnp.float32),
        scratch_types=[
            pltpu.VMEM((CW, W), jnp.int32),
            pltpu.VMEM((CW, W), jnp.int32),
            pltpu.VMEM((W, 128), jnp.float32),
            pltpu.VMEM_SHARED((NROWS, 128), jnp.float32),
        ],
    )
    def agg(h_hbm, init_hbm, src_hbm, dst_hbm, out_hbm, src_v, dst_v, rows_v, acc_sh):
        c = lax.axis_index("c")
        s = lax.axis_index("s")
        q = c * NSUB + s
        pltpu.sync_copy(init_hbm.at[q], acc_sh.at[pl.ds(s * rpt, rpt)])
        plsc.subcore_barrier()

        @pl.loop(0, NCHUNK1)
        def _(ch):
            pltpu.sync_copy(src_hbm.at[q, pl.ds(ch * CW, CW)], src_v)
            pltpu.sync_copy(dst_hbm.at[q, pl.ds(ch * CW, CW)], dst_v)

            @pl.loop(0, CW)
            def _(w):
                pltpu.sync_copy(h_hbm.at[src_v.at[w]], rows_v)
                pltpu.sync_copy(rows_v, acc_sh.at[dst_v.at[w]], add=True)

        plsc.subcore_barrier()
        pltpu.sync_copy(acc_sh.at[pl.ds(s * rpt, rpt)], out_hbm.at[q])

    def call(x, src1, dst1):
        init = jnp.concatenate(
            [x.reshape(NSUB, rpt, 128),
             jnp.zeros((NSUB, rpt, 128), jnp.float32)], axis=0)
        out = agg(x, init, src1, dst1)
        return out.reshape(2 * N, 128)

    return call


@functools.cache
def _make_agg(dh):
    mesh = plsc.VectorSubcoreMesh(core_axis_name="c", subcore_axis_name="s",
                                  num_cores=2, num_subcores=NSUB)
    rpt = N // NSUB

    @functools.partial(
        pl.kernel,
        mesh=mesh,
        out_type=jax.ShapeDtypeStruct((2 * NSUB, rpt, dh), jnp.float32),
        scratch_types=[
            pltpu.VMEM((CW, W), jnp.int32),
            pltpu.VMEM((CW, W), jnp.int32),
            pltpu.VMEM((W, dh), jnp.float32),
            pltpu.VMEM_SHARED((NROWS, dh), jnp.float32),
        ],
    )
    def agg(h_hbm, hblk_hbm, src_hbm, dst_hbm, out_hbm, src_v, dst_v, rows_v, acc_sh):
        c = lax.axis_index("c")
        s = lax.axis_index("s")
        q = c * NSUB + s
        pltpu.sync_copy(hblk_hbm.at[q], acc_sh.at[pl.ds(s * rpt, rpt)])
        plsc.subcore_barrier()

        @pl.loop(0, NCHUNK)
        def _(ch):
            pltpu.sync_copy(src_hbm.at[q, pl.ds(ch * CW, CW)], src_v)
            pltpu.sync_copy(dst_hbm.at[s, pl.ds(ch * CW, CW)], dst_v)

            @pl.loop(0, CW)
            def _(w):
                pltpu.sync_copy(h_hbm.at[src_v.at[w]], rows_v)
                pltpu.sync_copy(rows_v, acc_sh.at[dst_v.at[w]], add=True)

        plsc.subcore_barrier()
        pltpu.sync_copy(acc_sh.at[pl.ds(s * rpt, rpt)], out_hbm.at[q])

    def call(h, src2, dst3):
        out = agg(h, h.reshape(2 * NSUB, rpt, dh), src2, dst3)
        return out.reshape(2 * N, dh)

    return call


def _dot_bf16(a, b):
    return jnp.dot(a.astype(jnp.bfloat16), b.astype(jnp.bfloat16),
                   preferred_element_type=jnp.float32)


def _combine(hpa_ref, add):
    if add:
        return hpa_ref[0:N, :] + hpa_ref[N:2 * N, :]
    return jnp.concatenate([hpa_ref[0:N, :], hpa_ref[N:2 * N, :]], axis=1)


def _mlp_body(add, hpa_ref, w1_ref, b1_ref, w2_ref, b2_ref, r_ref):
    t = _combine(hpa_ref, add)
    z = jnp.maximum(_dot_bf16(t, w1_ref[...]) + b1_ref[...], 0.0)
    y = _dot_bf16(z, w2_ref[...]) + b2_ref[...]
    r_ref[...] = jnp.maximum(y, 0.0)


def _mlp_tc(hpa, w1, b1, w2, b2, add=False):
    return pl.pallas_call(
        functools.partial(_mlp_body, add),
        out_shape=jax.ShapeDtypeStruct((N, 256), jnp.float32),
    )(hpa, w1, b1.reshape(1, -1), w2, b2.reshape(1, -1))


def _norm_body(r_ref, m_ref, v_ref, g_ref, be_ref, out_ref):
    o = ((r_ref[...] - m_ref[...]) / jnp.sqrt(v_ref[...] + 1e-5)
         * g_ref[...] + be_ref[...])
    out_ref[0:N, :] = o[:, 0:128]
    out_ref[N:2 * N, :] = o[:, 128:256]


def _norm_tc(r, m, v, gamma, beta):
    return pl.pallas_call(
        _norm_body,
        out_shape=jax.ShapeDtypeStruct((2 * N, 128), jnp.float32),
    )(r, m.reshape(1, -1), v.reshape(1, -1),
      gamma.reshape(1, -1), beta.reshape(1, -1))


def _final_body(r_ref, m_ref, v_ref, g_ref, be_ref,
                batch_ref, f1w_ref, f1b_ref, f2w_ref, f2b_ref, out_ref):
    o = ((r_ref[...] - m_ref[...]) / jnp.sqrt(v_ref[...] + 1e-5)
         * g_ref[...] + be_ref[...])
    sel = batch_ref[...] == lax.broadcasted_iota(jnp.int32, (NG, N), 0)
    p = sel.astype(jnp.float32)
    g = jnp.dot(p, o, preferred_element_type=jnp.float32,
                precision=lax.Precision.HIGHEST)
    gf = jnp.maximum(_dot_bf16(g, f1w_ref[...]) + f1b_ref[...], 0.0)
    logits = _dot_bf16(gf, f2w_ref[...]) + f2b_ref[...]
    m = jnp.max(logits, axis=1, keepdims=True)
    lse = jnp.log(jnp.sum(jnp.exp(logits - m), axis=1, keepdims=True))
    out_ref[...] = (logits - m) - lse


def _final_tc(r, m, v, gamma, beta, batch2d, f1w, f1b, f2w, f2b):
    return pl.pallas_call(
        _final_body,
        out_shape=jax.ShapeDtypeStruct((NG, NCLS), jnp.float32),
    )(r, m.reshape(1, -1), v.reshape(1, -1),
      gamma.reshape(1, -1), beta.reshape(1, -1),
      batch2d, f1w, f1b.reshape(1, -1), f2w, f2b.reshape(1, -1))


def kernel(x, edge_index, batch,
           conv1_w1, conv1_b1, conv1_w2, conv1_b2,
           convs_w1, convs_b1, convs_w2, convs_b2,
           bn_gamma, bn_beta,
           fc1_w, fc1_b, fc2_w, fc2_b):
    def pad_tables(src, dst, npad):
        nt = src.shape[0]
        src_p = jnp.pad(src, ((0, 0), (0, npad)))
        lanes = jnp.arange(npad, dtype=jnp.int32)
        tiles = jnp.arange(nt, dtype=jnp.int32)
        trash = N + ((tiles[:, None] * npad + lanes[None, :]) % NTRASH)
        dst_p = jnp.concatenate([dst, trash], axis=1)
        return src_p, dst_p

    src_p, dst_p = pad_tables(edge_index[0].reshape(NSUB, EPT),
                              edge_index[1].reshape(NSUB, EPT), NPAD)
    src2 = jnp.stack([src_p, src_p + N]).reshape(2 * NSUB, NWIN, W)
    dst3 = dst_p.reshape(NSUB, NWIN, W)
    s1, d1 = pad_tables(edge_index[0].reshape(2 * NSUB, EPT1),
                        edge_index[1].reshape(2 * NSUB, EPT1), NPAD1)
    src1 = s1.reshape(2 * NSUB, NWIN1, W)
    dst1 = d1.reshape(2 * NSUB, NWIN1, W)
    batch2d = batch.reshape(1, N)

    hpa = _make_agg1()(x, src1, dst1)
    r = _mlp_tc(hpa, conv1_w1, conv1_b1, conv1_w2, conv1_b2, add=True)
    h = _norm_tc(r, jnp.mean(r, axis=0), jnp.var(r, axis=0),
                 bn_gamma[0], bn_beta[0])

    for i in range(3):
        hpa = _make_agg(128)(h, src2, dst3)
        r = _mlp_tc(hpa, convs_w1[i], convs_b1[i], convs_w2[i], convs_b2[i])
        h = _norm_tc(r, jnp.mean(r, axis=0), jnp.var(r, axis=0),
                     bn_gamma[i + 1], bn_beta[i + 1])

    hpa = _make_agg(128)(h, src2, dst3)
    r = _mlp_tc(hpa, convs_w1[3], convs_b1[3], convs_w2[3], convs_b2[3])
    return _final_tc(r, jnp.mean(r, axis=0), jnp.var(r, axis=0),
                     bn_gamma[4], bn_beta[4], batch2d,
                     fc1_w, fc1_b, fc2_w, fc2_b)


# speedup vs baseline: 3.5376x; 1.1947x over previous
import functools

import jax
import jax.numpy as jnp
from jax import lax
from jax.experimental import pallas as pl
from jax.experimental.pallas import tpu as pltpu
from jax.experimental.pallas import tpu_sc as plsc

N = 10000
E = 320000
NG = 64
NCLS = 10

NSUB = 16
EPT = E // NSUB
W = 128
CW = 16
NCHUNK = -(-EPT // (W * CW))
NWIN = NCHUNK * CW
EPT_PAD = NWIN * W
NPAD = EPT_PAD - EPT
NTRASH = 512
NROWS = N + NTRASH


EPT1 = E // (2 * NSUB)
NCHUNK1 = -(-EPT1 // (W * CW))
NWIN1 = NCHUNK1 * CW
NPAD1 = NWIN1 * W - EPT1


def _edge_loop(nchunk, qsrc, qdst, h_hbm, src_hbm, dst_hbm,
               src_v, dst_v, b0, b1, acc_sh, sem0, sem1):
    def g_start(w, buf, sem):
        pltpu.async_copy(h_hbm.at[src_v.at[w]], buf, sem)

    def g_wait(w, buf, sem):
        pltpu.make_async_copy(h_hbm.at[src_v.at[w]], buf, sem).wait()

    def s_add(w, buf):
        pltpu.sync_copy(buf, acc_sh.at[dst_v.at[w]], add=True)

    @pl.loop(0, nchunk)
    def _(ch):
        pltpu.sync_copy(src_hbm.at[qsrc, pl.ds(ch * CW, CW)], src_v)
        pltpu.sync_copy(dst_hbm.at[qdst, pl.ds(ch * CW, CW)], dst_v)
        g_start(0, b0, sem0)

        @pl.loop(0, CW // 2)
        def _(i):
            w0 = 2 * i
            g_start(w0 + 1, b1, sem1)
            g_wait(w0, b0, sem0)
            s_add(w0, b0)

            @pl.when(i < CW // 2 - 1)
            def _():
                g_start(w0 + 2, b0, sem0)

            g_wait(w0 + 1, b1, sem1)
            s_add(w0 + 1, b1)


@functools.cache
def _make_agg1():
    mesh = plsc.VectorSubcoreMesh(core_axis_name="c", subcore_axis_name="s",
                                  num_cores=2, num_subcores=NSUB)
    rpt = N // NSUB

    @functools.partial(
        pl.kernel,
        mesh=mesh,
        out_type=jax.ShapeDtypeStruct((2 * NSUB, rpt, 128), jnp.float32),
        scratch_types=[
            pltpu.VMEM((CW, W), jnp.int32),
            pltpu.VMEM((CW, W), jnp.int32),
            pltpu.VMEM((W, 128), jnp.float32),
            pltpu.VMEM((W, 128), jnp.float32),
            pltpu.VMEM_SHARED((NROWS, 128), jnp.float32),
            pltpu.SemaphoreType.DMA,
            pltpu.SemaphoreType.DMA,
        ],
    )
    def agg(h_hbm, init_hbm, src_hbm, dst_hbm, out_hbm,
            src_v, dst_v, b0, b1, acc_sh, sem0, sem1):
        c = lax.axis_index("c")
        s = lax.axis_index("s")
        q = c * NSUB + s
        pltpu.sync_copy(init_hbm.at[q], acc_sh.at[pl.ds(s * rpt, rpt)])
        plsc.subcore_barrier()

        _edge_loop(NCHUNK1, q, q, h_hbm, src_hbm, dst_hbm,
                   src_v, dst_v, b0, b1, acc_sh, sem0, sem1)

        plsc.subcore_barrier()
        pltpu.sync_copy(acc_sh.at[pl.ds(s * rpt, rpt)], out_hbm.at[q])

    def call(x, src1, dst1):
        init = jnp.concatenate(
            [x.reshape(NSUB, rpt, 128),
             jnp.zeros((NSUB, rpt, 128), jnp.float32)], axis=0)
        out = agg(x, init, src1, dst1)
        return out.reshape(2 * N, 128)

    return call


@functools.cache
def _make_agg(dh):
    mesh = plsc.VectorSubcoreMesh(core_axis_name="c", subcore_axis_name="s",
                                  num_cores=2, num_subcores=NSUB)
    rpt = N // NSUB

    @functools.partial(
        pl.kernel,
        mesh=mesh,
        out_type=jax.ShapeDtypeStruct((2 * NSUB, rpt, dh), jnp.float32),
        scratch_types=[
            pltpu.VMEM((CW, W), jnp.int32),
            pltpu.VMEM((CW, W), jnp.int32),
            pltpu.VMEM((W, dh), jnp.float32),
            pltpu.VMEM((W, dh), jnp.float32),
            pltpu.VMEM_SHARED((NROWS, dh), jnp.float32),
            pltpu.SemaphoreType.DMA,
            pltpu.SemaphoreType.DMA,
        ],
    )
    def agg(h_hbm, hblk_hbm, src_hbm, dst_hbm, out_hbm,
            src_v, dst_v, b0, b1, acc_sh, sem0, sem1):
        c = lax.axis_index("c")
        s = lax.axis_index("s")
        q = c * NSUB + s
        pltpu.sync_copy(hblk_hbm.at[q], acc_sh.at[pl.ds(s * rpt, rpt)])
        plsc.subcore_barrier()

        _edge_loop(NCHUNK, q, s, h_hbm, src_hbm, dst_hbm,
                   src_v, dst_v, b0, b1, acc_sh, sem0, sem1)

        plsc.subcore_barrier()
        pltpu.sync_copy(acc_sh.at[pl.ds(s * rpt, rpt)], out_hbm.at[q])

    def call(h, src2, dst3):
        out = agg(h, h.reshape(2 * NSUB, rpt, dh), src2, dst3)
        return out.reshape(2 * N, dh)

    return call


def _dot_bf16(a, b):
    return jnp.dot(a.astype(jnp.bfloat16), b.astype(jnp.bfloat16),
                   preferred_element_type=jnp.float32)


def _combine(hpa_ref, add):
    if add:
        return hpa_ref[0:N, :] + hpa_ref[N:2 * N, :]
    return jnp.concatenate([hpa_ref[0:N, :], hpa_ref[N:2 * N, :]], axis=1)


def _mlp_body(add, hpa_ref, w1_ref, b1_ref, w2_ref, b2_ref, r_ref):
    t = _combine(hpa_ref, add)
    z = jnp.maximum(_dot_bf16(t, w1_ref[...]) + b1_ref[...], 0.0)
    y = _dot_bf16(z, w2_ref[...]) + b2_ref[...]
    r_ref[...] = jnp.maximum(y, 0.0)


def _mlp_tc(hpa, w1, b1, w2, b2, add=False):
    return pl.pallas_call(
        functools.partial(_mlp_body, add),
        out_shape=jax.ShapeDtypeStruct((N, 256), jnp.float32),
    )(hpa, w1, b1.reshape(1, -1), w2, b2.reshape(1, -1))


def _norm_body(r_ref, m_ref, v_ref, g_ref, be_ref, out_ref):
    o = ((r_ref[...] - m_ref[...]) / jnp.sqrt(v_ref[...] + 1e-5)
         * g_ref[...] + be_ref[...])
    out_ref[0:N, :] = o[:, 0:128]
    out_ref[N:2 * N, :] = o[:, 128:256]


def _norm_tc(r, m, v, gamma, beta):
    return pl.pallas_call(
        _norm_body,
        out_shape=jax.ShapeDtypeStruct((2 * N, 128), jnp.float32),
    )(r, m.reshape(1, -1), v.reshape(1, -1),
      gamma.reshape(1, -1), beta.reshape(1, -1))


def _final_body(r_ref, m_ref, v_ref, g_ref, be_ref,
                batch_ref, f1w_ref, f1b_ref, f2w_ref, f2b_ref, out_ref):
    o = ((r_ref[...] - m_ref[...]) / jnp.sqrt(v_ref[...] + 1e-5)
         * g_ref[...] + be_ref[...])
    sel = batch_ref[...] == lax.broadcasted_iota(jnp.int32, (NG, N), 0)
    p = sel.astype(jnp.float32)
    g = jnp.dot(p, o, preferred_element_type=jnp.float32,
                precision=lax.Precision.HIGHEST)
    gf = jnp.maximum(_dot_bf16(g, f1w_ref[...]) + f1b_ref[...], 0.0)
    logits = _dot_bf16(gf, f2w_ref[...]) + f2b_ref[...]
    m = jnp.max(logits, axis=1, keepdims=True)
    lse = jnp.log(jnp.sum(jnp.exp(logits - m), axis=1, keepdims=True))
    out_ref[...] = (logits - m) - lse


def _final_tc(r, m, v, gamma, beta, batch2d, f1w, f1b, f2w, f2b):
    return pl.pallas_call(
        _final_body,
        out_shape=jax.ShapeDtypeStruct((NG, NCLS), jnp.float32),
    )(r, m.reshape(1, -1), v.reshape(1, -1),
      gamma.reshape(1, -1), beta.reshape(1, -1),
      batch2d, f1w, f1b.reshape(1, -1), f2w, f2b.reshape(1, -1))


def kernel(x, edge_index, batch,
           conv1_w1, conv1_b1, conv1_w2, conv1_b2,
           convs_w1, convs_b1, convs_w2, convs_b2,
           bn_gamma, bn_beta,
           fc1_w, fc1_b, fc2_w, fc2_b):
    def pad_tables(src, dst, npad):
        nt = src.shape[0]
        src_p = jnp.pad(src, ((0, 0), (0, npad)))
        lanes = jnp.arange(npad, dtype=jnp.int32)
        tiles = jnp.arange(nt, dtype=jnp.int32)
        trash = N + ((tiles[:, None] * npad + lanes[None, :]) % NTRASH)
        dst_p = jnp.concatenate([dst, trash], axis=1)
        return src_p, dst_p

    src_p, dst_p = pad_tables(edge_index[0].reshape(NSUB, EPT),
                              edge_index[1].reshape(NSUB, EPT), NPAD)
    src2 = jnp.stack([src_p, src_p + N]).reshape(2 * NSUB, NWIN, W)
    dst3 = dst_p.reshape(NSUB, NWIN, W)
    s1, d1 = pad_tables(edge_index[0].reshape(2 * NSUB, EPT1),
                        edge_index[1].reshape(2 * NSUB, EPT1), NPAD1)
    src1 = s1.reshape(2 * NSUB, NWIN1, W)
    dst1 = d1.reshape(2 * NSUB, NWIN1, W)
    batch2d = batch.reshape(1, N)

    hpa = _make_agg1()(x, src1, dst1)
    r = _mlp_tc(hpa, conv1_w1, conv1_b1, conv1_w2, conv1_b2, add=True)
    h = _norm_tc(r, jnp.mean(r, axis=0), jnp.var(r, axis=0),
                 bn_gamma[0], bn_beta[0])

    for i in range(3):
        hpa = _make_agg(128)(h, src2, dst3)
        r = _mlp_tc(hpa, convs_w1[i], convs_b1[i], convs_w2[i], convs_b2[i])
        h = _norm_tc(r, jnp.mean(r, axis=0), jnp.var(r, axis=0),
                     bn_gamma[i + 1], bn_beta[i + 1])

    hpa = _make_agg(128)(h, src2, dst3)
    r = _mlp_tc(hpa, convs_w1[3], convs_b1[3], convs_w2[3], convs_b2[3])
    return _final_tc(r, jnp.mean(r, axis=0), jnp.var(r, axis=0),
                     bn_gamma[4], bn_beta[4], batch2d,
                     fc1_w, fc1_b, fc2_w, fc2_b)
